# baseline (speedup 1.0000x reference)
import jax
import jax.numpy as jnp
from jax import lax
from jax.experimental import pallas as pl
from jax.experimental.pallas import tpu as pltpu
from jax.experimental.pallas import tpu_sc as plsc

SEQ_LEN = 8192
EMB = 1024
BSZ = 4

NUM_CORES = 2
NUM_SUBCORES = 16
NUM_WORKERS = NUM_CORES * NUM_SUBCORES
ROWS_PER_WORKER = SEQ_LEN // NUM_WORKERS
CHUNK = 32
NUM_CHUNKS = ROWS_PER_WORKER // CHUNK
NBUF = 2


def _sc_body(idx_hbm, table_hbm, out_hbm,
             idx_v, rows0, rows1, gsem0, gsem1, wsem0, wsem1):
    wid = lax.axis_index("s") * NUM_CORES + lax.axis_index("c")
    base = wid * ROWS_PER_WORKER
    bufs = (rows0, rows1)
    gsems = (gsem0, gsem1)
    wsems = (wsem0, wsem1)
    pltpu.sync_copy(idx_hbm.at[pl.ds(base, ROWS_PER_WORKER)], idx_v)
    pending_writes = {0: [], 1: []}
    for c in range(NUM_CHUNKS):
        k = c % NBUF
        for w in pending_writes[k]:
            w.wait()
        pending_writes[k] = []
        g = pltpu.async_copy(
            table_hbm.at[idx_v.at[pl.ds(c * CHUNK, CHUNK)]],
            bufs[k], gsems[k])
        g.wait()
        off = base + c * CHUNK
        for b in range(BSZ):
            pending_writes[k].append(pltpu.async_copy(
                bufs[k], out_hbm.at[b, pl.ds(off, CHUNK)], wsems[k]))
    for k in range(NBUF):
        for w in pending_writes[k]:
            w.wait()


def kernel(inputs, embedding_table, length=0):
    del inputs
    seq = jnp.arange(SEQ_LEN, dtype=jnp.int32) + jnp.asarray(
        length, dtype=jnp.int32)
    idx = jnp.clip(seq, 0, SEQ_LEN - 1)
    mesh = plsc.VectorSubcoreMesh(
        core_axis_name="c", subcore_axis_name="s")
    run = pl.kernel(
        _sc_body,
        out_type=jax.ShapeDtypeStruct((BSZ, SEQ_LEN, EMB), jnp.float32),
        mesh=mesh,
        scratch_types=[
            pltpu.VMEM((ROWS_PER_WORKER,), jnp.int32),
            pltpu.VMEM((CHUNK, EMB), jnp.float32),
            pltpu.VMEM((CHUNK, EMB), jnp.float32),
            pltpu.SemaphoreType.DMA,
            pltpu.SemaphoreType.DMA,
            pltpu.SemaphoreType.DMA,
            pltpu.SemaphoreType.DMA,
        ],
    )
    return run(idx, embedding_table)


# speedup vs baseline: 1.0391x; 1.0391x over previous
import jax
import jax.numpy as jnp
from jax import lax
from jax.experimental import pallas as pl
from jax.experimental.pallas import tpu as pltpu
from jax.experimental.pallas import tpu_sc as plsc

SEQ_LEN = 8192
EMB = 1024
BSZ = 4

NUM_CORES = 2
NUM_SUBCORES = 16
NUM_WORKERS = NUM_CORES * NUM_SUBCORES
ROWS_PER_WORKER = SEQ_LEN // NUM_WORKERS
CHUNK = 56
CHUNKS = [56, 56, 56, 56, 32]
NBUF = 2


def _sc_body(idx_hbm, table_hbm, out_hbm,
             idx_v, rows0, rows1, gsem0, gsem1, wsem0, wsem1):
    wid = lax.axis_index("s") * NUM_CORES + lax.axis_index("c")
    base = wid * ROWS_PER_WORKER
    bufs = (rows0, rows1)
    gsems = (gsem0, gsem1)
    wsems = (wsem0, wsem1)
    pltpu.sync_copy(idx_hbm.at[pl.ds(base, ROWS_PER_WORKER)], idx_v)
    pending_writes = {0: [], 1: []}
    off = 0
    for c, n in enumerate(CHUNKS):
        k = c % NBUF
        for w in pending_writes[k]:
            w.wait()
        pending_writes[k] = []
        dst = bufs[k] if n == CHUNK else bufs[k].at[pl.ds(0, n)]
        pltpu.async_copy(
            table_hbm.at[idx_v.at[pl.ds(off, n)]], dst, gsems[k]).wait()
        for b in range(BSZ):
            pending_writes[k].append(pltpu.async_copy(
                dst, out_hbm.at[b, pl.ds(base + off, n)], wsems[k]))
        off += n
    for k in range(NBUF):
        for w in pending_writes[k]:
            w.wait()


def kernel(inputs, embedding_table, length=0):
    del inputs
    seq = jnp.arange(SEQ_LEN, dtype=jnp.int32) + jnp.asarray(
        length, dtype=jnp.int32)
    idx = jnp.clip(seq, 0, SEQ_LEN - 1)
    mesh = plsc.VectorSubcoreMesh(
        core_axis_name="c", subcore_axis_name="s")
    run = pl.kernel(
        _sc_body,
        out_type=jax.ShapeDtypeStruct((BSZ, SEQ_LEN, EMB), jnp.float32),
        mesh=mesh,
        scratch_types=[
            pltpu.VMEM((ROWS_PER_WORKER,), jnp.int32),
            pltpu.VMEM((CHUNK, EMB), jnp.float32),
            pltpu.VMEM((CHUNK, EMB), jnp.float32),
            pltpu.SemaphoreType.DMA,
            pltpu.SemaphoreType.DMA,
            pltpu.SemaphoreType.DMA,
            pltpu.SemaphoreType.DMA,
        ],
    )
    return run(idx, embedding_table)
